# baseline (speedup 1.0000x reference)
import functools

import jax
import jax.numpy as jnp
from jax.experimental import pallas as pl
from jax.experimental.pallas import tpu as pltpu

_NPOINTS = [2048, 512, 128, 32]
_NSAMPLE = 32


def _rup(x, m):
    return ((x + m - 1) // m) * m


def _mlp_body(nlayers, final_act, x_ref, *refs):
    out_ref = refs[-1]
    h = x_ref[0]
    for i in range(nlayers):
        w = refs[2 * i][...]
        b = refs[2 * i + 1][...]
        h = jnp.dot(w, h, preferred_element_type=jnp.float32) + b
        if i < nlayers - 1 or final_act:
            h = jnp.maximum(h, 0.0)
    out_ref[0] = h


def _mlp_stack(x, ws, bs, final_act=True, tile_n=512):
    B, Cin, N = x.shape
    Np = _rup(N, 128)
    tn = min(tile_n, Np)
    Np = _rup(Np, tn)
    if Np != N:
        x = jnp.pad(x, ((0, 0), (0, 0), (0, Np - N)))
    nlayers = len(ws)
    Cout = ws[-1].shape[0]
    in_specs = [pl.BlockSpec((1, Cin, tn), lambda b, n: (b, 0, n))]
    args = [x]
    for w, b in zip(ws, bs):
        in_specs.append(pl.BlockSpec(w.shape, lambda b, n: (0, 0)))
        in_specs.append(pl.BlockSpec((w.shape[0], 1), lambda b, n: (0, 0)))
        args.append(w)
        args.append(b.reshape(-1, 1))
    out = pl.pallas_call(
        functools.partial(_mlp_body, nlayers, final_act),
        grid=(B, Np // tn),
        in_specs=in_specs,
        out_specs=pl.BlockSpec((1, Cout, tn), lambda b, n: (b, 0, n)),
        out_shape=jax.ShapeDtypeStruct((B, Cout, Np), jnp.float32),
    )(*args)
    return out[:, :, :N] if Np != N else out


def _knn(query, ref, k):
    q2 = jnp.sum(query * query, -1, keepdims=True)
    r2 = jnp.sum(ref * ref, -1)[:, None, :]
    d = q2 + r2 - 2.0 * jnp.einsum('bmd,bnd->bmn', query, ref)
    negd, idx = jax.lax.top_k(-d, k)
    return -negd, idx


def _gather_max(h, idx):
    g = jax.vmap(lambda f, i: jnp.take(f, i, axis=1))(h, idx)
    return jnp.max(g, axis=-1)


def _gather_interp(h, idx, w):
    g = jax.vmap(lambda f, i: jnp.take(f, i, axis=1))(h, idx)
    return jnp.sum(g * w[:, None, :, :], axis=-1)


def kernel(pointcloud, conv0_ws, conv0_bs, sa_ws, sa_bs, fp_ws, fp_bs, cls_ws, cls_bs):
    xyz = pointcloud[..., 0:3]
    feats = jnp.transpose(pointcloud[..., 3:], (0, 2, 1))

    h = _mlp_stack(feats, conv0_ws, conv0_bs)
    _, idx0 = _knn(xyz, xyz, _NSAMPLE)
    f0 = _gather_max(h, idx0)

    l_xyz = [xyz]
    l_feats = [f0]
    for k in range(4):
        q = l_xyz[k][:, :_NPOINTS[k]]
        _, idx = _knn(q, l_xyz[k], _NSAMPLE)
        h = _mlp_stack(l_feats[k], sa_ws[k], sa_bs[k])
        l_xyz.append(q)
        l_feats.append(_gather_max(h, idx))

    for i in range(-1, -5, -1):
        d, idx = _knn(l_xyz[i - 1], l_xyz[i], 3)
        d = jnp.maximum(d, 1e-10)
        w = 1.0 / d
        w = w / jnp.sum(w, -1, keepdims=True)
        interp = _gather_interp(l_feats[i], idx, w)
        x = jnp.concatenate([interp, l_feats[i - 1]], axis=1)
        l_feats[i - 1] = _mlp_stack(x, fp_ws[i], fp_bs[i])

    pred = _mlp_stack(l_feats[0], list(cls_ws), list(cls_bs), final_act=False)
    return jnp.transpose(pred, (0, 2, 1))
